# baseline (speedup 1.0000x reference)
import functools

import jax
import jax.numpy as jnp
from jax import lax
from jax.experimental import pallas as pl
from jax.experimental.pallas import tpu as pltpu
from jax.experimental.pallas import tpu_sc as plsc

_RO = 4.0
_B, _P, _N = 2, 32, 128
_C = 64
_S = 128
_D, _H, _W = 32, 64, 64
_NPTS = _B * _P * _N
_TOT = _NPTS * 8


_IBLK = 1024


def _corner_math(pts, row0, nrows, idx_ref, w_ref):
    z = jnp.clip(pts[:, 0:1], 0.0, _D - 1 - 1e-4)
    y = jnp.clip(pts[:, 1:2], 0.0, _H - 1 - 1e-4)
    x = jnp.clip(pts[:, 2:3], 0.0, _W - 1 - 1e-4)
    z0 = jnp.floor(z)
    y0 = jnp.floor(y)
    x0 = jnp.floor(x)
    z0i = z0.astype(jnp.int32)
    y0i = y0.astype(jnp.int32)
    x0i = x0.astype(jnp.int32)
    z1i = jnp.minimum(z0i + 1, _D - 1)
    y1i = jnp.minimum(y0i + 1, _H - 1)
    x1i = jnp.minimum(x0i + 1, _W - 1)
    wz = z - z0
    wy = y - y0
    wx = x - x0
    b = ((row0 + jax.lax.broadcasted_iota(jnp.int32, (nrows, 1), 0))
         // (_NPTS // _B))
    base = b * (_D * _H * _W)
    for k in range(8):
        dz, dy, dx = (k >> 2) & 1, (k >> 1) & 1, k & 1
        zi = z1i if dz else z0i
        yi = y1i if dy else y0i
        xi = x1i if dx else x0i
        idx_ref[:, k:k + 1] = base + (zi * _H + yi) * _W + xi
        w_ref[:, k:k + 1] = ((wz if dz else 1.0 - wz)
                             * (wy if dy else 1.0 - wy)
                             * (wx if dx else 1.0 - wx))


def _idx_kernel(pts_ref, idx_ref, w_ref):
    _corner_math(pts_ref[...], pl.program_id(0) * _IBLK, _IBLK, idx_ref, w_ref)


def _corner_indices(pts):
    return pl.pallas_call(
        _idx_kernel,
        grid=(_NPTS // _IBLK,),
        in_specs=[pl.BlockSpec((_IBLK, 3), lambda i: (i, 0))],
        out_specs=(pl.BlockSpec((_IBLK, 8), lambda i: (i, 0)),
                   pl.BlockSpec((_IBLK, 8), lambda i: (i, 0))),
        out_shape=(jax.ShapeDtypeStruct((_NPTS, 8), jnp.int32),
                   jax.ShapeDtypeStruct((_NPTS, 8), jnp.float32)),
    )(pts)


_NW = 32
_PER_W = _TOT // _NW
_CH = 128
_NCH = _PER_W // _CH
_GPB = 4

@functools.cache
def _build_sc_gather():
    mesh = plsc.VectorSubcoreMesh(core_axis_name="c", subcore_axis_name="s")

    @functools.partial(
        pl.kernel,
        mesh=mesh,
        out_type=jax.ShapeDtypeStruct((_TOT, _C), jnp.float32),
        scratch_types=[
            pltpu.VMEM((_NCH, _CH), jnp.int32),
            pltpu.VMEM((_GPB * _CH, _C), jnp.float32),
            pltpu.VMEM((_GPB * _CH, _C), jnp.float32),
            pltpu.SemaphoreType.DMA,
            pltpu.SemaphoreType.DMA,
        ],
        compiler_params=pltpu.CompilerParams(use_tc_tiling_on_sc=False),
    )
    def _sc_gather(table_hbm, idx_hbm, out_hbm, idx_v, buf0, buf1, gsem, osem):
        wid = lax.axis_index("s") * 2 + lax.axis_index("c")
        pltpu.sync_copy(idx_hbm.at[pl.ds(wid * _NCH, _NCH)], idx_v)
        bufs = (buf0, buf1)
        ngrp = _NCH // _GPB
        out_cp = [None, None]
        for g in range(ngrp):
            buf = bufs[g % 2]
            if g >= 2:
                out_cp[g % 2].wait()
            pend = [
                pltpu.async_copy(table_hbm.at[idx_v.at[g * _GPB + j]],
                                 buf.at[pl.ds(j * _CH, _CH)], gsem)
                for j in range(_GPB)
            ]
            for h in pend:
                h.wait()
            out_cp[g % 2] = pltpu.async_copy(
                buf,
                out_hbm.at[pl.ds(wid * _PER_W + g * _GPB * _CH, _GPB * _CH)],
                osem)
        out_cp[0].wait()
        out_cp[1].wait()

    return _sc_gather


def _gather_rows(table, idx2d):
    return _build_sc_gather()(table, idx2d)


_GBLK = 16
_RBLK = _GBLK * _N
_GRID = (_NPTS // _RBLK,)


def _roll_n(hg, s):
    return jnp.concatenate([hg[:, -s % _N:, :], hg[:, : -s % _N, :]], axis=1)


def _gcn_kernel(pts_ref, w8_ref, gath_ref, wct_ref, wcot_ref, bh_ref,
                wlt_ref, bl_ref, wpt_ref, bp_ref,
                out_ref, nidx_ref, nw_ref):
    pts = pts_ref[...]
    w8 = w8_ref[...]
    feat = w8[:, 0:1] * gath_ref[:, 0:_C]
    for k in range(1, 8):
        feat = feat + w8[:, k:k + 1] * gath_ref[:, _C * k:_C * (k + 1)]
    pg = pts.reshape(_GBLK, _N, 3)
    pmin = jnp.min(pg, axis=1, keepdims=True)
    can = (pg - pmin).reshape(_RBLK, 3)
    coord = jnp.concatenate([pts[:, 0:1], can[:, 1:3] * _RO], axis=1)
    h = (jnp.dot(feat, wct_ref[...], preferred_element_type=jnp.float32)
         + jnp.dot(coord, wcot_ref[...], preferred_element_type=jnp.float32)
         + bh_ref[...])
    h = jnp.maximum(h, 0.0)
    for l in range(4):
        hg = h.reshape(_GBLK, _N, _S)
        agg = (_roll_n(hg, 1) + _roll_n(hg, -1)
               + _roll_n(hg, 2) + _roll_n(hg, -2)) * 0.25
        hin = h + agg.reshape(_RBLK, _S)
        h = jnp.maximum(
            jnp.dot(hin, wlt_ref[l], preferred_element_type=jnp.float32)
            + bl_ref[l:l + 1, :], 0.0) + h
    off = (jnp.dot(h, wpt_ref[...], preferred_element_type=jnp.float32)
           + bp_ref[...])
    pyro = jnp.concatenate(
        [pts[:, 0:1] * (1.0 / _RO), pts[:, 1:3] + off * (1.0 / _RO)], axis=1)
    out_ref[...] = pyro
    _corner_math(pyro, pl.program_id(0) * _RBLK, _RBLK, nidx_ref, nw_ref)


def _gcn_stage(pts, w8, gath2, wct, wcot, bh, wlt, bl, wpt, bp):
    def full(shape):
        return pl.BlockSpec(shape, lambda i: tuple(0 for _ in shape))
    return pl.pallas_call(
        _gcn_kernel,
        grid=_GRID,
        in_specs=[
            pl.BlockSpec((_RBLK, 3), lambda i: (i, 0)),
            pl.BlockSpec((_RBLK, 8), lambda i: (i, 0)),
            pl.BlockSpec((_RBLK, 8 * _C), lambda i: (i, 0)),
            full((_C, _S)),
            full((3, _S)),
            full((1, _S)),
            full((4, _S, _S)),
            full((4, _S)),
            full((_S, 2)),
            full((1, 2)),
        ],
        out_specs=(pl.BlockSpec((_RBLK, 3), lambda i: (i, 0)),
                   pl.BlockSpec((_RBLK, 8), lambda i: (i, 0)),
                   pl.BlockSpec((_RBLK, 8), lambda i: (i, 0))),
        out_shape=(jax.ShapeDtypeStruct((_NPTS, 3), jnp.float32),
                   jax.ShapeDtypeStruct((_NPTS, 8), jnp.int32),
                   jax.ShapeDtypeStruct((_NPTS, 8), jnp.float32)),
    )(pts, w8, gath2, wct, wcot, bh, wlt, bl, wpt, bp)


def _prep_params(p):
    wct = p['W_head'][:, :_C].T
    wcot = p['W_head'][:, _C:].T
    bh = p['b_head'][None, :]
    wlt = jnp.stack([w.T for w in p['W_layers']])
    bl = jnp.stack(p['b_layers'])
    wpt = p['W_pred'].T
    bp = p['b_pred'][None, :]
    return wct, wcot, bh, wlt, bl, wpt, bp


def kernel(cnn_feature, init_contour, params):
    table = cnn_feature.transpose(0, 2, 3, 4, 1).reshape(_B * _D * _H * _W, _C)
    pts = init_contour.reshape(_NPTS, 3)
    idx8, w8 = _corner_indices(pts)
    outs = []
    for name in ('resgcn', 'resgcn0', 'resgcn1'):
        gath = _gather_rows(table, idx8.reshape(_TOT // _CH, _CH))
        gath2 = gath.reshape(_NPTS, 8 * _C)
        pts, idx8, w8 = _gcn_stage(pts, w8, gath2,
                                   *_prep_params(params[name]))
        outs.append(pts)
    return jnp.stack(outs, axis=0).reshape(3, _B, _P, _N, 3)


# speedup vs baseline: 1.2068x; 1.2068x over previous
import functools

import jax
import jax.numpy as jnp
from jax import lax
from jax.experimental import pallas as pl
from jax.experimental.pallas import tpu as pltpu
from jax.experimental.pallas import tpu_sc as plsc

_RO = 4.0
_B, _P, _N = 2, 32, 128
_C = 64
_S = 128
_D, _H, _W = 32, 64, 64
_NPTS = _B * _P * _N
_TOT = _NPTS * 8


_IBLK = 1024


def _corner_math(pts, row0, nrows, idx_ref, w_ref):
    z = jnp.clip(pts[:, 0:1], 0.0, _D - 1 - 1e-4)
    y = jnp.clip(pts[:, 1:2], 0.0, _H - 1 - 1e-4)
    x = jnp.clip(pts[:, 2:3], 0.0, _W - 1 - 1e-4)
    z0 = jnp.floor(z)
    y0 = jnp.floor(y)
    x0 = jnp.floor(x)
    z0i = z0.astype(jnp.int32)
    y0i = y0.astype(jnp.int32)
    x0i = x0.astype(jnp.int32)
    z1i = jnp.minimum(z0i + 1, _D - 1)
    y1i = jnp.minimum(y0i + 1, _H - 1)
    x1i = jnp.minimum(x0i + 1, _W - 1)
    wz = z - z0
    wy = y - y0
    wx = x - x0
    b = ((row0 + jax.lax.broadcasted_iota(jnp.int32, (nrows, 1), 0))
         // (_NPTS // _B))
    base = b * (_D * _H * _W)
    for k in range(8):
        dz, dy, dx = (k >> 2) & 1, (k >> 1) & 1, k & 1
        zi = z1i if dz else z0i
        yi = y1i if dy else y0i
        xi = x1i if dx else x0i
        idx_ref[:, k:k + 1] = base + (zi * _H + yi) * _W + xi
        w_ref[:, k:k + 1] = ((wz if dz else 1.0 - wz)
                             * (wy if dy else 1.0 - wy)
                             * (wx if dx else 1.0 - wx))


def _idx_kernel(pts_ref, idx_ref, w_ref):
    _corner_math(pts_ref[...], pl.program_id(0) * _IBLK, _IBLK, idx_ref, w_ref)


def _corner_indices(pts):
    return pl.pallas_call(
        _idx_kernel,
        grid=(_NPTS // _IBLK,),
        in_specs=[pl.BlockSpec((_IBLK, 3), lambda i: (i, 0))],
        out_specs=(pl.BlockSpec((_IBLK, 8), lambda i: (i, 0)),
                   pl.BlockSpec((_IBLK, 8), lambda i: (i, 0))),
        out_shape=(jax.ShapeDtypeStruct((_NPTS, 8), jnp.int32),
                   jax.ShapeDtypeStruct((_NPTS, 8), jnp.float32)),
    )(pts)


_NW = 32
_PER_W = _TOT // _NW
_CH = 128
_NCH = _PER_W // _CH
_GPB = 4

_PPW = _NPTS // _NW
_CPTS = 16
_NCHK = _PPW // _CPTS


def _splat16(v):
    return jax.lax.broadcast_in_dim(jnp.int32(v), (16,), ())


@functools.cache
def _build_sc_gather():
    mesh = plsc.VectorSubcoreMesh(core_axis_name="c", subcore_axis_name="s")

    @functools.partial(
        pl.kernel,
        mesh=mesh,
        out_type=jax.ShapeDtypeStruct((_NPTS, _C), jnp.float32),
        scratch_types=[
            pltpu.VMEM((_NCHK, _CPTS * 8), jnp.int32),
            pltpu.VMEM((_PPW * 8,), jnp.float32),
            pltpu.VMEM((_CPTS * 8, _C), jnp.float32),
            pltpu.VMEM((_CPTS * 8, _C), jnp.float32),
            pltpu.VMEM((_PPW, _C), jnp.float32),
            pltpu.SemaphoreType.DMA,
            pltpu.SemaphoreType.DMA,
            pltpu.SemaphoreType.DMA,
        ],
        compiler_params=pltpu.CompilerParams(use_tc_tiling_on_sc=False,
                                             needs_layout_passes=False),
    )
    def _sc_gather(table_hbm, idx_hbm, w_hbm, out_hbm,
                   idx_v, w_v, buf_a, buf_b, feat_v, sem_a, sem_b, osem):
        wid = lax.axis_index("s") * 2 + lax.axis_index("c")
        pbase = wid * _PPW
        pltpu.sync_copy(idx_hbm.at[pl.ds(wid * _NCHK, _NCHK)], idx_v)
        pltpu.sync_copy(w_hbm.at[pl.ds(pbase * 8, _PPW * 8)], w_v)

        def fire(chunk, buf, sem):
            return pltpu.async_copy(
                table_hbm.at[idx_v.at[chunk]], buf, sem)

        def gwait(chunk, buf, sem):
            pltpu.make_async_copy(
                table_hbm.at[idx_v.at[chunk]], buf, sem).wait()

        def compute(chunk, buf):
            for j in range(_CPTS):
                accs = [None] * (_C // 16)
                for k in range(8):
                    wv = plsc.load_gather(
                        w_v, [_splat16((chunk * _CPTS + j) * 8 + k)])
                    for q in range(_C // 16):
                        v = buf[j * 8 + k, pl.ds(q * 16, 16)]
                        accs[q] = wv * v if k == 0 else accs[q] + wv * v
                for q in range(_C // 16):
                    feat_v[chunk * _CPTS + j, pl.ds(q * 16, 16)] = accs[q]

        fire(0, buf_a, sem_a)

        def body(i, carry):
            a = 2 * i
            hb = fire(a + 1, buf_b, sem_b)
            gwait(a, buf_a, sem_a)
            compute(a, buf_a)
            @pl.when(i < _NCHK // 2 - 1)
            def _():
                fire(a + 2, buf_a, sem_a)
            hb.wait()
            compute(a + 1, buf_b)
            return carry

        lax.fori_loop(0, _NCHK // 2, body, 0)
        pltpu.async_copy(feat_v, out_hbm.at[pl.ds(pbase, _PPW)], osem).wait()

    return _sc_gather


def _gather_rows(table, idx8, w8):
    return _build_sc_gather()(table, idx8.reshape(_TOT // 128, 128),
                              w8.reshape(_TOT))


_GBLK = 16
_RBLK = _GBLK * _N
_GRID = (_NPTS // _RBLK,)


def _roll_n(hg, s):
    return jnp.concatenate([hg[:, -s % _N:, :], hg[:, : -s % _N, :]], axis=1)


def _gcn_kernel(pts_ref, feat_ref, wct_ref, wcot_ref, bh_ref,
                wlt_ref, bl_ref, wpt_ref, bp_ref, out_ref):
    pts = pts_ref[...]
    feat = feat_ref[...]
    pg = pts.reshape(_GBLK, _N, 3)
    pmin = jnp.min(pg, axis=1, keepdims=True)
    can = (pg - pmin).reshape(_RBLK, 3)
    coord = jnp.concatenate([pts[:, 0:1], can[:, 1:3] * _RO], axis=1)
    h = (jnp.dot(feat, wct_ref[...], preferred_element_type=jnp.float32)
         + jnp.dot(coord, wcot_ref[...], preferred_element_type=jnp.float32)
         + bh_ref[...])
    h = jnp.maximum(h, 0.0)
    for l in range(4):
        hg = h.reshape(_GBLK, _N, _S)
        agg = (_roll_n(hg, 1) + _roll_n(hg, -1)
               + _roll_n(hg, 2) + _roll_n(hg, -2)) * 0.25
        hin = h + agg.reshape(_RBLK, _S)
        h = jnp.maximum(
            jnp.dot(hin, wlt_ref[l], preferred_element_type=jnp.float32)
            + bl_ref[l:l + 1, :], 0.0) + h
    off = (jnp.dot(h, wpt_ref[...], preferred_element_type=jnp.float32)
           + bp_ref[...])
    out_ref[...] = jnp.concatenate(
        [pts[:, 0:1] * (1.0 / _RO), pts[:, 1:3] + off * (1.0 / _RO)], axis=1)


def _gcn_stage(pts, feat, wct, wcot, bh, wlt, bl, wpt, bp):
    def full(shape):
        return pl.BlockSpec(shape, lambda i: tuple(0 for _ in shape))
    return pl.pallas_call(
        _gcn_kernel,
        grid=_GRID,
        in_specs=[
            pl.BlockSpec((_RBLK, 3), lambda i: (i, 0)),
            pl.BlockSpec((_RBLK, _C), lambda i: (i, 0)),
            full((_C, _S)),
            full((3, _S)),
            full((1, _S)),
            full((4, _S, _S)),
            full((4, _S)),
            full((_S, 2)),
            full((1, 2)),
        ],
        out_specs=pl.BlockSpec((_RBLK, 3), lambda i: (i, 0)),
        out_shape=jax.ShapeDtypeStruct((_NPTS, 3), jnp.float32),
    )(pts, feat, wct, wcot, bh, wlt, bl, wpt, bp)


def _prep_params(p):
    wct = p['W_head'][:, :_C].T
    wcot = p['W_head'][:, _C:].T
    bh = p['b_head'][None, :]
    wlt = jnp.stack([w.T for w in p['W_layers']])
    bl = jnp.stack(p['b_layers'])
    wpt = p['W_pred'].T
    bp = p['b_pred'][None, :]
    return wct, wcot, bh, wlt, bl, wpt, bp


def kernel(cnn_feature, init_contour, params):
    table = cnn_feature.transpose(0, 2, 3, 4, 1).reshape(_B * _D * _H * _W, _C)
    pts = init_contour.reshape(_NPTS, 3)
    outs = []
    for name in ('resgcn', 'resgcn0', 'resgcn1'):
        idx8, w8 = _corner_indices(pts)
        feat = _gather_rows(table, idx8, w8)
        pts = _gcn_stage(pts, feat, *_prep_params(params[name]))
        outs.append(pts)
    return jnp.stack(outs, axis=0).reshape(3, _B, _P, _N, 3)
